# baseline (speedup 1.0000x reference)
import jax
import jax.numpy as jnp
from jax.experimental import pallas as pl
from jax.experimental.pallas import tpu as pltpu

_BM = 256


def _masked_matmul_kernel(a_ref, m_ref, b_ref, o_ref):
    a_blk = a_ref[0] * m_ref[...]
    o_ref[0] = jnp.dot(a_blk, b_ref[0], preferred_element_type=jnp.float32)


def kernel(a, mask, b):
    B, H, S, _ = a.shape
    D = b.shape[-1]
    a3 = a.reshape(H, S, S)
    m2 = mask.reshape(S, S).astype(jnp.float32)
    b3 = b.reshape(H, S, D)

    grid = (S // _BM, H)
    out = pl.pallas_call(
        _masked_matmul_kernel,
        grid=grid,
        in_specs=[
            pl.BlockSpec((1, _BM, S), lambda i, h: (h, i, 0)),
            pl.BlockSpec((_BM, S), lambda i, h: (i, 0)),
            pl.BlockSpec((1, S, D), lambda i, h: (h, 0, 0)),
        ],
        out_specs=pl.BlockSpec((1, _BM, D), lambda i, h: (h, i, 0)),
        out_shape=jax.ShapeDtypeStruct((H, S, D), jnp.float32),
        compiler_params=pltpu.CompilerParams(
            dimension_semantics=("arbitrary", "arbitrary"),
        ),
    )(a3, m2, b3)
    return out.reshape(B, H, S, D)


# speedup vs baseline: 1.1141x; 1.1141x over previous
import jax
import jax.numpy as jnp
from jax.experimental import pallas as pl
from jax.experimental.pallas import tpu as pltpu

_BM = 256


def _masked_matmul_kernel(a_ref, m_ref, b_ref, o_ref):
    h = pl.program_id(1)
    a_blk = a_ref[0] * m_ref[...].astype(jnp.float32)
    o_ref[0] = jnp.dot(a_blk, b_ref[h], preferred_element_type=jnp.float32)


def kernel(a, mask, b):
    B, H, S, _ = a.shape
    D = b.shape[-1]
    a3 = a.reshape(H, S, S)
    m2 = mask.reshape(S, S)
    b3 = b.reshape(H, S, D)

    grid = (S // _BM, H)
    out = pl.pallas_call(
        _masked_matmul_kernel,
        grid=grid,
        in_specs=[
            pl.BlockSpec((1, _BM, S), lambda i, h: (h, i, 0)),
            pl.BlockSpec((_BM, S), lambda i, h: (i, 0)),
            pl.BlockSpec((H, S, D), lambda i, h: (0, 0, 0)),
        ],
        out_specs=pl.BlockSpec((1, _BM, D), lambda i, h: (h, i, 0)),
        out_shape=jax.ShapeDtypeStruct((H, S, D), jnp.float32),
        compiler_params=pltpu.CompilerParams(
            dimension_semantics=("arbitrary", "arbitrary"),
        ),
    )(a3, m2, b3)
    return out.reshape(B, H, S, D)


# speedup vs baseline: 1.1143x; 1.0001x over previous
import jax
import jax.numpy as jnp
from jax.experimental import pallas as pl
from jax.experimental.pallas import tpu as pltpu

_BM = 256


def _masked_matmul_kernel(a_ref, m_ref, b_ref, o_ref):
    i = pl.program_id(1)
    m_blk = m_ref[pl.ds(i * _BM, _BM), :].astype(jnp.float32)
    o_ref[0] = jnp.dot(a_ref[0] * m_blk, b_ref[0],
                       preferred_element_type=jnp.float32)


def kernel(a, mask, b):
    B, H, S, _ = a.shape
    D = b.shape[-1]
    a3 = a.reshape(H, S, S)
    m2 = mask.reshape(S, S)
    b3 = b.reshape(H, S, D)

    grid = (H, S // _BM)
    out = pl.pallas_call(
        _masked_matmul_kernel,
        grid=grid,
        in_specs=[
            pl.BlockSpec((1, _BM, S), lambda h, i: (h, i, 0)),
            pl.BlockSpec((S, S), lambda h, i: (0, 0)),
            pl.BlockSpec((1, S, D), lambda h, i: (h, 0, 0)),
        ],
        out_specs=pl.BlockSpec((1, _BM, D), lambda h, i: (h, i, 0)),
        out_shape=jax.ShapeDtypeStruct((H, S, D), jnp.float32),
        compiler_params=pltpu.CompilerParams(
            dimension_semantics=("arbitrary", "arbitrary"),
        ),
    )(a3, m2, b3)
    return out.reshape(B, H, S, D)


# speedup vs baseline: 1.3588x; 1.2194x over previous
import jax
import jax.numpy as jnp
from jax.experimental import pallas as pl
from jax.experimental.pallas import tpu as pltpu

_BM = 512


def _masked_matmul_kernel(a_ref, m_ref, b_ref, o_ref):
    i = pl.program_id(1)
    m_blk = m_ref[pl.ds(i * _BM, _BM), :].astype(jnp.float32)
    o_ref[0] = jnp.dot(a_ref[0] * m_blk, b_ref[0],
                       preferred_element_type=jnp.float32)


def kernel(a, mask, b):
    B, H, S, _ = a.shape
    D = b.shape[-1]
    a3 = a.reshape(H, S, S)
    m2 = mask.reshape(S, S)
    b3 = b.reshape(H, S, D)

    grid = (H, S // _BM)
    out = pl.pallas_call(
        _masked_matmul_kernel,
        grid=grid,
        in_specs=[
            pl.BlockSpec((1, _BM, S), lambda h, i: (h, i, 0)),
            pl.BlockSpec((S, S), lambda h, i: (0, 0)),
            pl.BlockSpec((1, S, D), lambda h, i: (h, 0, 0)),
        ],
        out_specs=pl.BlockSpec((1, _BM, D), lambda h, i: (h, i, 0)),
        out_shape=jax.ShapeDtypeStruct((H, S, D), jnp.float32),
        compiler_params=pltpu.CompilerParams(
            dimension_semantics=("arbitrary", "arbitrary"),
        ),
    )(a3, m2, b3)
    return out.reshape(B, H, S, D)


# speedup vs baseline: 1.4643x; 1.0776x over previous
import jax
import jax.numpy as jnp
from jax.experimental import pallas as pl
from jax.experimental.pallas import tpu as pltpu

_BM = 1024


def _masked_matmul_kernel(a_ref, m_ref, b_ref, o_ref):
    i = pl.program_id(1)
    m_blk = m_ref[pl.ds(i * _BM, _BM), :].astype(jnp.float32)
    o_ref[0] = jnp.dot(a_ref[0] * m_blk, b_ref[0],
                       preferred_element_type=jnp.float32)


def kernel(a, mask, b):
    B, H, S, _ = a.shape
    D = b.shape[-1]
    a3 = a.reshape(H, S, S)
    m2 = mask.reshape(S, S)
    b3 = b.reshape(H, S, D)

    grid = (H, S // _BM)
    out = pl.pallas_call(
        _masked_matmul_kernel,
        grid=grid,
        in_specs=[
            pl.BlockSpec((1, _BM, S), lambda h, i: (h, i, 0)),
            pl.BlockSpec((S, S), lambda h, i: (0, 0)),
            pl.BlockSpec((1, S, D), lambda h, i: (h, 0, 0)),
        ],
        out_specs=pl.BlockSpec((1, _BM, D), lambda h, i: (h, i, 0)),
        out_shape=jax.ShapeDtypeStruct((H, S, D), jnp.float32),
        compiler_params=pltpu.CompilerParams(
            dimension_semantics=("arbitrary", "arbitrary"),
        ),
    )(a3, m2, b3)
    return out.reshape(B, H, S, D)
